# Initial kernel scaffold; baseline (speedup 1.0000x reference)
import jax
import jax.numpy as jnp
from jax.experimental import pallas as pl


def kernel(x_num, x_cat, cat_tables, num_W, num_b):
    raise NotImplementedError("write your pallas kernel here")


# speedup vs baseline: 1.0364x; 1.0364x over previous
import functools

import jax
import jax.numpy as jnp
from jax import lax
from jax.experimental import pallas as pl
from jax.experimental.pallas import tpu as pltpu
from jax.experimental.pallas import tpu_sc as plsc

B = 16384
FC = 26
FN = 13
V = 100000
D = 16
FT = FC + FN

NC = 2
NS = 16
NW = NC * NS
ROWS_PER_W = B // NW
CB = 128
NSUB = ROWS_PER_W // CB


def _body(xnum_hbm, xcatt_hbm, tab_hbm, w_hbm, b_hbm, out_hbm,
          xcat_v, rows_v, xnum_v, numout_v, wv, bv, gsem, osem):
    c = lax.axis_index("c")
    s = lax.axis_index("s")
    wid = s * NC + c
    base = wid * ROWS_PER_W

    pltpu.sync_copy(w_hbm, wv)
    pltpu.sync_copy(b_hbm, bv)
    wrows = [wv[j] for j in range(FN)]
    brows = [bv[j] for j in range(FN)]

    for sub in range(NSUB):
        b0 = base + sub * CB

        pltpu.sync_copy(xcatt_hbm.at[:, pl.ds(b0, CB)], xcat_v)

        for f in range(FC):
            off = f * V
            for i in range(CB // 16):
                sl = pl.ds(i * 16, 16)
                xcat_v[f, sl] = xcat_v[f, sl] + off

        gcps = []
        for f in range(FC):
            gcps.append(
                pltpu.async_copy(tab_hbm.at[xcat_v.at[f]], rows_v.at[f], gsem))

        pltpu.sync_copy(xnum_hbm.at[pl.ds(b0, CB)], xnum_v)

        def nbody(i, carry):
            xsrow = xnum_v[i]
            for j in range(FN):
                numout_v[i, j, :] = (
                    jnp.broadcast_to(xsrow[j], (D,)) * wrows[j] + brows[j])
            return carry

        lax.fori_loop(0, CB, nbody, 0)
        ncp = pltpu.async_copy(
            numout_v, out_hbm.at[pl.ds(b0, CB), pl.ds(FC, FN), :], osem)

        ocps = []
        for f in range(FC):
            gcps[f].wait()
            ocps.append(
                pltpu.async_copy(rows_v.at[f], out_hbm.at[pl.ds(b0, CB), f, :],
                                 osem))
        ncp.wait()
        for o in ocps:
            o.wait()


@jax.jit
def _tokenize(x_num, xcat_t, tab_flat, num_W, num_b):
    k = pl.kernel(
        _body,
        out_type=jax.ShapeDtypeStruct((B, FT, D), jnp.float32),
        mesh=plsc.VectorSubcoreMesh(core_axis_name="c", subcore_axis_name="s"),
        compiler_params=pltpu.CompilerParams(use_tc_tiling_on_sc=False),
        scratch_types=[
            pltpu.VMEM((FC, CB), jnp.int32),
            pltpu.VMEM((FC, CB, D), jnp.float32),
            pltpu.VMEM((CB, 16), jnp.float32),
            pltpu.VMEM((CB, FN, D), jnp.float32),
            pltpu.VMEM((FN, D), jnp.float32),
            pltpu.VMEM((FN, D), jnp.float32),
            pltpu.SemaphoreType.DMA,
            pltpu.SemaphoreType.DMA,
        ],
    )
    return k(x_num, xcat_t, tab_flat, num_W, num_b)


def kernel(x_num, x_cat, cat_tables, num_W, num_b):
    xcat_t = x_cat.astype(jnp.int32).T
    tab_flat = cat_tables.reshape(FC * V, D)
    xnum_p = jnp.pad(x_num, ((0, 0), (0, 16 - FN)))
    return _tokenize(xnum_p, xcat_t, tab_flat, num_W, num_b)
